# baseline (speedup 1.0000x reference)
import functools

import jax
import jax.numpy as jnp
from jax import lax
from jax.experimental import pallas as pl
from jax.experimental.pallas import tpu as pltpu
from jax.experimental.pallas import tpu_sc as plsc

NC = 2
NS = 16
NW = NC * NS
L = 16
CH = 16384

_MAGIC = 12582912.0
_MAGIC_I = 1262485504


def _index16(x):
    f = (x - 0.5) + _MAGIC
    i = jax.lax.bitcast_convert_type(f, jnp.int32) - (_MAGIC_I - 8)
    return jnp.minimum(jnp.maximum(i, 0), 15)


def _sc_body(x_hbm, idx_hbm, x_v0, x_v1, idx_v0, idx_v1,
             sin0, sin1, sout0, sout1, n_ref):
    wid = lax.axis_index("s") * NC + lax.axis_index("c")
    per_w = n_ref[0] // NW
    chunks = per_w // CH
    base0 = wid * per_w

    x_v = (x_v0, x_v1)
    idx_v = (idx_v0, idx_v1)
    sin = (sin0, sin1)
    sout = (sout0, sout1)

    def in_copy(g, b):
        return pltpu.make_async_copy(
            x_hbm.at[pl.ds(base0 + g * CH, CH)], x_v[b], sin[b])

    def out_copy(g, b):
        return pltpu.make_async_copy(
            idx_v[b], idx_hbm.at[pl.ds(base0 + g * CH, CH)], sout[b])

    in_copy(0, 0).start()
    for g in range(chunks):
        b = g % 2
        if g + 1 < chunks:
            in_copy(g + 1, 1 - b).start()
        in_copy(g, b).wait()
        if g >= 2:
            out_copy(g - 2, b).wait()

        @plsc.parallel_loop(0, CH, step=L, unroll=16)
        def _vec_body(i):
            idx_v[b][pl.ds(i, L)] = _index16(x_v[b][pl.ds(i, L)])

        out_copy(g, b).start()
    for g in (chunks - 2, chunks - 1):
        if g >= 0:
            out_copy(g, g % 2).wait()


def _tc_vals_body(x_ref, vals_ref):
    x = x_ref[...]
    vals_ref[...] = jnp.ceil(jnp.clip(x, -7.0, 8.0)) - 0.5


@jax.jit
def _quantize(x_flat):
    n = x_flat.shape[0]
    assert n % (NW * CH) == 0

    mesh = plsc.VectorSubcoreMesh(
        core_axis_name="c", subcore_axis_name="s", num_cores=NC, num_subcores=NS
    )
    idx = pl.kernel(
        functools.partial(_sc_body, n_ref=(n,)),
        out_type=jax.ShapeDtypeStruct((n,), jnp.int32),
        mesh=mesh,
        scratch_types=[
            pltpu.VMEM((CH,), jnp.float32),
            pltpu.VMEM((CH,), jnp.float32),
            pltpu.VMEM((CH,), jnp.int32),
            pltpu.VMEM((CH,), jnp.int32),
            pltpu.SemaphoreType.DMA,
            pltpu.SemaphoreType.DMA,
            pltpu.SemaphoreType.DMA,
            pltpu.SemaphoreType.DMA,
        ],
    )(x_flat)

    blk = 1024 * 1024
    vals = pl.pallas_call(
        _tc_vals_body,
        grid=(n // blk,),
        in_specs=[pl.BlockSpec((blk,), lambda r: (r,))],
        out_specs=pl.BlockSpec((blk,), lambda r: (r,)),
        out_shape=jax.ShapeDtypeStruct((n,), jnp.float32),
    )(x_flat)

    return vals, idx


def kernel(X, grid, grid_norm):
    vals, idx = _quantize(X.reshape(-1))
    return vals.reshape(-1, 1), idx


# speedup vs baseline: 1.0191x; 1.0191x over previous
import functools

import jax
import jax.numpy as jnp
from jax import lax
from jax.experimental import pallas as pl
from jax.experimental.pallas import tpu as pltpu
from jax.experimental.pallas import tpu_sc as plsc

NC = 2
NS = 16
NW = NC * NS
L = 16
CH = 16384
NB = 3

_MAGIC = 12582912.0
_MAGIC_I = 1262485504


def _quantize16(x):
    f = (x - 0.5) + _MAGIC
    i = jax.lax.bitcast_convert_type(f, jnp.int32) - (_MAGIC_I - 8)
    idx = jnp.minimum(jnp.maximum(i, 0), 15)
    vals = idx.astype(jnp.float32) - 7.5
    return vals, idx


def _sc_body(x_hbm, vals_hbm, idx_hbm, *scratch, n):
    x_v = scratch[0:NB]
    vals_v = scratch[NB:NB + 2]
    idx_v = scratch[NB + 2:NB + 4]
    sin = scratch[NB + 4:2 * NB + 4]
    sout = scratch[2 * NB + 4:2 * NB + 6]

    wid = lax.axis_index("s") * NC + lax.axis_index("c")
    per_w = n // NW
    chunks = per_w // CH
    base0 = wid * per_w

    def in_copy(g):
        return pltpu.make_async_copy(
            x_hbm.at[pl.ds(base0 + g * CH, CH)], x_v[g % NB], sin[g % NB])

    def out_copies(g):
        base = base0 + g * CH
        b = g % 2
        return (
            pltpu.make_async_copy(vals_v[b], vals_hbm.at[pl.ds(base, CH)],
                                  sout[b]),
            pltpu.make_async_copy(idx_v[b], idx_hbm.at[pl.ds(base, CH)],
                                  sout[b]),
        )

    for g in range(min(NB - 1, chunks)):
        in_copy(g).start()
    for g in range(chunks):
        bx = g % NB
        bo = g % 2
        if g + NB - 1 < chunks:
            in_copy(g + NB - 1).start()
        in_copy(g).wait()
        if g >= 2:
            for c in out_copies(g - 2):
                c.wait()

        @plsc.parallel_loop(0, CH, step=L, unroll=8)
        def _vec_body(i):
            vals, idx = _quantize16(x_v[bx][pl.ds(i, L)])
            vals_v[bo][pl.ds(i, L)] = vals
            idx_v[bo][pl.ds(i, L)] = idx

        for c in out_copies(g):
            c.start()
    for g in range(max(0, chunks - 2), chunks):
        for c in out_copies(g):
            c.wait()


@jax.jit
def _sc_quantize(x_flat):
    n = x_flat.shape[0]
    assert n % (NW * CH) == 0
    mesh = plsc.VectorSubcoreMesh(
        core_axis_name="c", subcore_axis_name="s", num_cores=NC, num_subcores=NS
    )
    f = pl.kernel(
        functools.partial(_sc_body, n=n),
        out_type=(
            jax.ShapeDtypeStruct((n,), jnp.float32),
            jax.ShapeDtypeStruct((n,), jnp.int32),
        ),
        mesh=mesh,
        scratch_types=(
            [pltpu.VMEM((CH,), jnp.float32) for _ in range(NB)]
            + [pltpu.VMEM((CH,), jnp.float32) for _ in range(2)]
            + [pltpu.VMEM((CH,), jnp.int32) for _ in range(2)]
            + [pltpu.SemaphoreType.DMA for _ in range(NB)]
            + [pltpu.SemaphoreType.DMA for _ in range(2)]
        ),
    )
    return f(x_flat)


def kernel(X, grid, grid_norm):
    vals, idx = _sc_quantize(X.reshape(-1))
    return vals.reshape(-1, 1), idx
